# baseline (speedup 1.0000x reference)
import functools

import jax
import jax.numpy as jnp
from jax.experimental import pallas as pl
from jax.experimental.pallas import tpu as pltpu

N = 10000
M_BLK = 200


def _gcn_body(x_ref, w_ref, b_ref, adj_ref, out_ref, supp_ref):
    @pl.when(pl.program_id(0) == 0)
    def _():
        supp_ref[...] = jnp.dot(
            x_ref[...], w_ref[...], preferred_element_type=jnp.float32
        )

    acc = jnp.dot(adj_ref[...], supp_ref[...], preferred_element_type=jnp.float32)
    out_ref[...] = jnp.maximum(acc + b_ref[...], 0.0)


@jax.jit
def kernel(x, adj, W, b):
    n, nfeat = x.shape
    nhid = W.shape[1]
    grid = (n // M_BLK,)
    return pl.pallas_call(
        _gcn_body,
        grid=grid,
        in_specs=[
            pl.BlockSpec((n, nfeat), lambda i: (0, 0)),
            pl.BlockSpec((nfeat, nhid), lambda i: (0, 0)),
            pl.BlockSpec((1, nhid), lambda i: (0, 0)),
            pl.BlockSpec((M_BLK, N), lambda i: (i, 0)),
        ],
        out_specs=pl.BlockSpec((M_BLK, nhid), lambda i: (i, 0)),
        out_shape=jax.ShapeDtypeStruct((n, nhid), jnp.float32),
        scratch_shapes=[pltpu.VMEM((N, nhid), jnp.float32)],
    )(x, W, b.reshape(1, nhid), adj)
